# baseline (speedup 1.0000x reference)
import functools

import jax
import jax.numpy as jnp
from jax import lax
from jax.experimental import pallas as pl
from jax.experimental.pallas import tpu as pltpu
from jax.experimental.pallas import tpu_sc as plsc

_NC = 2
_NS = 16
_NW = _NC * _NS
_L = 16
_R_SC = 128
_BT = 128


def _sc_call(x, pos_table, T, H):
    rows_w = _R_SC // _NW
    row0_sc = T - _R_SC
    mesh = plsc.VectorSubcoreMesh(core_axis_name="c", subcore_axis_name="s")

    @functools.partial(
        pl.kernel,
        out_type=jax.ShapeDtypeStruct((_R_SC, H), jnp.float32),
        mesh=mesh,
        scratch_types=[
            pltpu.VMEM((H,), jnp.float32),
            pltpu.VMEM((rows_w, H), jnp.float32),
            pltpu.SemaphoreType.DMA,
        ],
    )
    def sc_add(x_hbm, pos_hbm, out_hbm, xv, buf, sem):
        wid = lax.axis_index("s") * _NC + lax.axis_index("c")
        base = wid * rows_w
        pltpu.sync_copy(x_hbm, xv)
        gather = pltpu.make_async_copy(
            pos_hbm.at[pl.ds(row0_sc + base, rows_w)], buf, sem)
        gather.start()
        gather.wait()

        def panel_body(jp, carry):
            col0 = jp * (8 * _L)
            xvals = [xv[pl.ds(col0 + k * _L, _L)] for k in range(8)]
            for r in range(rows_w):
                for k in range(8):
                    plsc.addupdate(buf.at[r, pl.ds(col0 + k * _L, _L)],
                                   xvals[k])
            return carry

        lax.fori_loop(0, H // (8 * _L), panel_body, 0)
        pltpu.sync_copy(buf, out_hbm.at[pl.ds(base, rows_w)])

    return sc_add(x.reshape(T), pos_table)


def _tc_body(x_ref, pos_ref, o_ref):
    o_ref[...] = pos_ref[...] + x_ref[...]


def kernel(x, pos_table):
    B, T = x.shape
    H = pos_table.shape[1]
    r_tc = T - _R_SC

    sc_out = _sc_call(x, pos_table, T, H)

    tc_out = pl.pallas_call(
        _tc_body,
        grid=(r_tc // _BT,),
        in_specs=[
            pl.BlockSpec((1, H), lambda i: (0, 0)),
            pl.BlockSpec((_BT, H), lambda i: (i, 0)),
        ],
        out_specs=pl.BlockSpec((_BT, H), lambda i: (i, 0)),
        out_shape=jax.ShapeDtypeStruct((T, H), jnp.float32),
    )(x, pos_table)

    out = lax.dynamic_update_slice(tc_out, sc_out, (r_tc, 0))
    return out[None]


# speedup vs baseline: 1.1351x; 1.1351x over previous
import functools

import jax
import jax.numpy as jnp
from jax import lax
from jax.experimental import pallas as pl
from jax.experimental.pallas import tpu as pltpu
from jax.experimental.pallas import tpu_sc as plsc

_NC = 2
_NS = 16
_NW = _NC * _NS
_L = 16
_R_SC = 128
_BT = 384


def _sc_call(x, pos_table, T, H):
    rows_w = _R_SC // _NW
    row0_sc = T - _R_SC
    mesh = plsc.VectorSubcoreMesh(core_axis_name="c", subcore_axis_name="s")

    @functools.partial(
        pl.kernel,
        out_type=jax.ShapeDtypeStruct((_R_SC, H), jnp.float32),
        mesh=mesh,
        scratch_types=[
            pltpu.VMEM((H,), jnp.float32),
            pltpu.VMEM((rows_w, H), jnp.float32),
            pltpu.SemaphoreType.DMA,
        ],
    )
    def sc_add(x_hbm, pos_hbm, out_hbm, xv, buf, sem):
        wid = lax.axis_index("s") * _NC + lax.axis_index("c")
        base = wid * rows_w
        pltpu.sync_copy(x_hbm, xv)
        gather = pltpu.make_async_copy(
            pos_hbm.at[pl.ds(row0_sc + base, rows_w)], buf, sem)
        gather.start()
        gather.wait()

        def panel_body(jp, carry):
            col0 = jp * (8 * _L)
            xvals = [xv[pl.ds(col0 + k * _L, _L)] for k in range(8)]
            for r in range(rows_w):
                for k in range(8):
                    plsc.addupdate(buf.at[r, pl.ds(col0 + k * _L, _L)],
                                   xvals[k])
            return carry

        lax.fori_loop(0, H // (8 * _L), panel_body, 0)
        pltpu.sync_copy(buf, out_hbm.at[pl.ds(base, rows_w)])

    return sc_add(x.reshape(T), pos_table)


def _tc_body(x_ref, pos_ref, o_ref):
    o_ref[...] = pos_ref[...] + x_ref[...]


def kernel(x, pos_table):
    B, T = x.shape
    H = pos_table.shape[1]
    r_tc = T - _R_SC

    sc_out = _sc_call(x, pos_table, T, H)

    tc_out = pl.pallas_call(
        _tc_body,
        grid=(r_tc // _BT,),
        in_specs=[
            pl.BlockSpec((1, H), lambda i: (0, 0)),
            pl.BlockSpec((_BT, H), lambda i: (i, 0)),
        ],
        out_specs=pl.BlockSpec((_BT, H), lambda i: (i, 0)),
        out_shape=jax.ShapeDtypeStruct((T, H), jnp.float32),
    )(x, pos_table)

    out = lax.dynamic_update_slice(tc_out, sc_out, (r_tc, 0))
    return out[None]


# speedup vs baseline: 1.1486x; 1.0119x over previous
import functools

import jax
import jax.numpy as jnp
from jax import lax
from jax.experimental import pallas as pl
from jax.experimental.pallas import tpu as pltpu
from jax.experimental.pallas import tpu_sc as plsc

_NC = 2
_NS = 16
_NW = _NC * _NS
_L = 16
_R_SC = 128
_BT = 640


def _sc_call(x, pos_table, T, H):
    rows_w = _R_SC // _NW
    row0_sc = T - _R_SC
    mesh = plsc.VectorSubcoreMesh(core_axis_name="c", subcore_axis_name="s")

    @functools.partial(
        pl.kernel,
        out_type=jax.ShapeDtypeStruct((_R_SC, H), jnp.float32),
        mesh=mesh,
        scratch_types=[
            pltpu.VMEM((H,), jnp.float32),
            pltpu.VMEM((rows_w, H), jnp.float32),
            pltpu.SemaphoreType.DMA,
        ],
    )
    def sc_add(x_hbm, pos_hbm, out_hbm, xv, buf, sem):
        wid = lax.axis_index("s") * _NC + lax.axis_index("c")
        base = wid * rows_w
        pltpu.sync_copy(x_hbm, xv)
        gather = pltpu.make_async_copy(
            pos_hbm.at[pl.ds(row0_sc + base, rows_w)], buf, sem)
        gather.start()
        gather.wait()

        def panel_body(jp, carry):
            col0 = jp * (8 * _L)
            xvals = [xv[pl.ds(col0 + k * _L, _L)] for k in range(8)]
            for r in range(rows_w):
                for k in range(8):
                    plsc.addupdate(buf.at[r, pl.ds(col0 + k * _L, _L)],
                                   xvals[k])
            return carry

        lax.fori_loop(0, H // (8 * _L), panel_body, 0)
        pltpu.sync_copy(buf, out_hbm.at[pl.ds(base, rows_w)])

    return sc_add(x.reshape(T), pos_table)


def _tc_body(x_ref, pos_ref, o_ref):
    o_ref[...] = pos_ref[...] + x_ref[...]


def kernel(x, pos_table):
    B, T = x.shape
    H = pos_table.shape[1]
    r_tc = T - _R_SC

    sc_out = _sc_call(x, pos_table, T, H)

    tc_out = pl.pallas_call(
        _tc_body,
        grid=(r_tc // _BT,),
        in_specs=[
            pl.BlockSpec((1, H), lambda i: (0, 0)),
            pl.BlockSpec((_BT, H), lambda i: (i, 0)),
        ],
        out_specs=pl.BlockSpec((_BT, H), lambda i: (i, 0)),
        out_shape=jax.ShapeDtypeStruct((T, H), jnp.float32),
    )(x, pos_table)

    out = lax.dynamic_update_slice(tc_out, sc_out, (r_tc, 0))
    return out[None]


# speedup vs baseline: 1.1624x; 1.0120x over previous
import functools

import jax
import jax.numpy as jnp
from jax import lax
from jax.experimental import pallas as pl
from jax.experimental.pallas import tpu as pltpu
from jax.experimental.pallas import tpu_sc as plsc

_NC = 2
_NS = 16
_NW = _NC * _NS
_L = 16
_R_SC = 64
_BT = 496


def _sc_call(x, pos_table, T, H):
    rows_w = _R_SC // _NW
    row0_sc = T - _R_SC
    mesh = plsc.VectorSubcoreMesh(core_axis_name="c", subcore_axis_name="s")

    @functools.partial(
        pl.kernel,
        out_type=jax.ShapeDtypeStruct((_R_SC, H), jnp.float32),
        mesh=mesh,
        scratch_types=[
            pltpu.VMEM((H,), jnp.float32),
            pltpu.VMEM((rows_w, H), jnp.float32),
            pltpu.SemaphoreType.DMA,
        ],
    )
    def sc_add(x_hbm, pos_hbm, out_hbm, xv, buf, sem):
        wid = lax.axis_index("s") * _NC + lax.axis_index("c")
        base = wid * rows_w
        pltpu.sync_copy(x_hbm, xv)
        gather = pltpu.make_async_copy(
            pos_hbm.at[pl.ds(row0_sc + base, rows_w)], buf, sem)
        gather.start()
        gather.wait()

        def panel_body(jp, carry):
            col0 = jp * (8 * _L)
            xvals = [xv[pl.ds(col0 + k * _L, _L)] for k in range(8)]
            for r in range(rows_w):
                for k in range(8):
                    plsc.addupdate(buf.at[r, pl.ds(col0 + k * _L, _L)],
                                   xvals[k])
            return carry

        lax.fori_loop(0, H // (8 * _L), panel_body, 0)
        pltpu.sync_copy(buf, out_hbm.at[pl.ds(base, rows_w)])

    return sc_add(x.reshape(T), pos_table)


def _tc_body(x_ref, pos_ref, o_ref):
    o_ref[...] = pos_ref[...] + x_ref[...]


def kernel(x, pos_table):
    B, T = x.shape
    H = pos_table.shape[1]
    r_tc = T - _R_SC

    sc_out = _sc_call(x, pos_table, T, H)

    tc_out = pl.pallas_call(
        _tc_body,
        grid=(r_tc // _BT,),
        in_specs=[
            pl.BlockSpec((1, H), lambda i: (0, 0)),
            pl.BlockSpec((_BT, H), lambda i: (i, 0)),
        ],
        out_specs=pl.BlockSpec((_BT, H), lambda i: (i, 0)),
        out_shape=jax.ShapeDtypeStruct((T, H), jnp.float32),
    )(x, pos_table)

    out = lax.dynamic_update_slice(tc_out, sc_out, (r_tc, 0))
    return out[None]


# speedup vs baseline: 1.2286x; 1.0570x over previous
import functools

import jax
import jax.numpy as jnp
from jax import lax
from jax.experimental import pallas as pl
from jax.experimental.pallas import tpu as pltpu
from jax.experimental.pallas import tpu_sc as plsc

_NC = 2
_NS = 16
_NW = _NC * _NS
_L = 16
_R_SC = 64
_BT = 496


def _sc_call(x, pos_table, T, H):
    rows_w = _R_SC // _NS
    row0_sc = T - _R_SC
    mesh = plsc.VectorSubcoreMesh(core_axis_name="c", subcore_axis_name="s",
                                  num_cores=1)

    @functools.partial(
        pl.kernel,
        out_type=jax.ShapeDtypeStruct((_R_SC, H), jnp.float32),
        mesh=mesh,
        scratch_types=[
            pltpu.VMEM((H,), jnp.float32),
            pltpu.VMEM((rows_w, H), jnp.float32),
            pltpu.SemaphoreType.DMA,
        ],
    )
    def sc_add(x_hbm, pos_hbm, out_hbm, xv, buf, sem):
        wid = lax.axis_index("s")
        base = wid * rows_w
        pltpu.sync_copy(x_hbm, xv)
        gather = pltpu.make_async_copy(
            pos_hbm.at[pl.ds(row0_sc + base, rows_w)], buf, sem)
        gather.start()
        gather.wait()

        def panel_body(jp, carry):
            col0 = jp * (8 * _L)
            xvals = [xv[pl.ds(col0 + k * _L, _L)] for k in range(8)]
            for r in range(rows_w):
                for k in range(8):
                    plsc.addupdate(buf.at[r, pl.ds(col0 + k * _L, _L)],
                                   xvals[k])
            return carry

        lax.fori_loop(0, H // (8 * _L), panel_body, 0)
        pltpu.sync_copy(buf, out_hbm.at[pl.ds(base, rows_w)])

    return sc_add(x.reshape(T), pos_table)


def _tc_body(x_ref, pos_ref, o_ref):
    o_ref[...] = pos_ref[...] + x_ref[...]


def kernel(x, pos_table):
    B, T = x.shape
    H = pos_table.shape[1]
    r_tc = T - _R_SC

    sc_out = _sc_call(x, pos_table, T, H)

    tc_out = pl.pallas_call(
        _tc_body,
        grid=(r_tc // _BT,),
        in_specs=[
            pl.BlockSpec((1, H), lambda i: (0, 0)),
            pl.BlockSpec((_BT, H), lambda i: (i, 0)),
        ],
        out_specs=pl.BlockSpec((_BT, H), lambda i: (i, 0)),
        out_shape=jax.ShapeDtypeStruct((T, H), jnp.float32),
    )(x, pos_table)

    out = lax.dynamic_update_slice(tc_out, sc_out, (r_tc, 0))
    return out[None]


# speedup vs baseline: 1.2399x; 1.0091x over previous
import functools

import jax
import jax.numpy as jnp
from jax import lax
from jax.experimental import pallas as pl
from jax.experimental.pallas import tpu as pltpu
from jax.experimental.pallas import tpu_sc as plsc

_NC = 2
_NS = 16
_NW = _NC * _NS
_L = 16
_R_SC = 32
_BT = 504


def _sc_call(x, pos_table, T, H):
    rows_w = _R_SC // _NS
    row0_sc = T - _R_SC
    mesh = plsc.VectorSubcoreMesh(core_axis_name="c", subcore_axis_name="s",
                                  num_cores=1)

    @functools.partial(
        pl.kernel,
        out_type=jax.ShapeDtypeStruct((_R_SC, H), jnp.float32),
        mesh=mesh,
        scratch_types=[
            pltpu.VMEM((H,), jnp.float32),
            pltpu.VMEM((rows_w, H), jnp.float32),
        ],
    )
    def sc_add(x_hbm, pos_hbm, out_hbm, xv, buf):
        wid = lax.axis_index("s")
        base = wid * rows_w
        pltpu.sync_copy(x_hbm, xv)
        pltpu.sync_copy(pos_hbm.at[pl.ds(row0_sc + base, rows_w)], buf)

        def panel_body(jp, carry):
            col0 = jp * (8 * _L)
            xvals = [xv[pl.ds(col0 + k * _L, _L)] for k in range(8)]
            for r in range(rows_w):
                for k in range(8):
                    plsc.addupdate(buf.at[r, pl.ds(col0 + k * _L, _L)],
                                   xvals[k])
            return carry

        lax.fori_loop(0, H // (8 * _L), panel_body, 0)
        pltpu.sync_copy(buf, out_hbm.at[pl.ds(base, rows_w)])

    return sc_add(x.reshape(T), pos_table)


def _tc_body(x_ref, pos_ref, o_ref):
    o_ref[...] = pos_ref[...] + x_ref[...]


def kernel(x, pos_table):
    B, T = x.shape
    H = pos_table.shape[1]
    r_tc = T - _R_SC

    sc_out = _sc_call(x, pos_table, T, H)

    tc_out = pl.pallas_call(
        _tc_body,
        grid=(r_tc // _BT,),
        in_specs=[
            pl.BlockSpec((1, H), lambda i: (0, 0)),
            pl.BlockSpec((_BT, H), lambda i: (i, 0)),
        ],
        out_specs=pl.BlockSpec((_BT, H), lambda i: (i, 0)),
        out_shape=jax.ShapeDtypeStruct((T, H), jnp.float32),
    )(x, pos_table)

    out = lax.dynamic_update_slice(tc_out, sc_out, (r_tc, 0))
    return out[None]
